# baseline (speedup 1.0000x reference)
import functools

import jax
import jax.numpy as jnp
from jax import lax
from jax.experimental import pallas as pl
from jax.experimental.pallas import tpu as pltpu
from jax.experimental.pallas import tpu_sc as plsc

N = 10000
F_IN = 128
H = 256
HH = 128
T_OUT = 10
E = 320000
NS = 16
NC = 2
EPT = E // NS
CH = 80
NCHK = EPT // CH
EPW = E // (NS * NC)
DNCHK = EPW // CH

NH = 1496
NPASS = 7
NHA = 1536
AZR = NHA // NS
TRASH = NH

_mesh = plsc.VectorSubcoreMesh(core_axis_name="c", subcore_axis_name="s")


@functools.partial(
    pl.kernel,
    out_type=jax.ShapeDtypeStruct((NC * N, HH), jnp.float32),
    mesh=_mesh,
    scratch_types=[
        pltpu.VMEM((NCHK, CH), jnp.int32),
        pltpu.VMEM((NCHK, CH), jnp.int32),
        pltpu.VMEM((EPT,), jnp.float32),
        pltpu.VMEM((CH, HH), jnp.float32),
        pltpu.VMEM((AZR, HH), jnp.float32),
        pltpu.VMEM((CH,), jnp.int32),
        pltpu.VMEM_SHARED((NHA, HH), jnp.float32),
        pltpu.SemaphoreType.DMA,
    ],
)
def _sc_agg(xs_flat, src_w, dst_w, ew_w, out, srcv, dstv, ewv, rows, zbuf,
            dst2, acc, sem):
  c = lax.axis_index("c")
  s = lax.axis_index("s")
  pltpu.sync_copy(src_w.at[s], srcv)
  pltpu.sync_copy(dst_w.at[s], dstv)
  pltpu.sync_copy(ew_w.at[s], ewv)

  zeros16 = jnp.zeros((16,), jnp.float32)

  def zb_body(i, _):
    for f in range(HH // 16):
      zbuf[i, pl.ds(f * 16, 16)] = zeros16
    return 0

  lax.fori_loop(0, AZR, zb_body, 0, unroll=2)

  cbase = c * (xs_flat.shape[0] // 2)

  def off_body(i, _):
    for m in range(CH // 16):
      sl = pl.ds(m * 16, 16)
      srcv[i, sl] = srcv[i, sl] + cbase
    return 0

  lax.fori_loop(0, NCHK, off_body, 0)

  for p in range(NPASS):
    pltpu.sync_copy(zbuf, acc.at[pl.ds(s * AZR, AZR)])
    plsc.subcore_barrier()

    lo = p * NH

    def chunk_body(k, _):
      pltpu.async_copy(xs_flat.at[srcv.at[k]], rows, sem).wait()

      def edge_body(m, _):
        w16 = ewv[pl.ds(k * CH + m * 16, 16)]
        sl16 = pl.ds(m * 16, 16)
        d16 = dstv[k, sl16] - lo
        ok = (d16 >= 0) & (d16 < NH)
        dst2[sl16] = jnp.where(ok, d16, TRASH)
        for j in range(16):
          e = m * 16 + j
          w = w16[j]
          for f in range(HH // 16):
            sl = pl.ds(f * 16, 16)
            rows[e, sl] = rows[e, sl] * w
        return 0

      lax.fori_loop(0, CH // 16, edge_body, 0)
      pltpu.sync_copy(rows, acc.at[dst2], add=True)
      return 0

    lax.fori_loop(0, NCHK, chunk_body, 0)
    plsc.subcore_barrier()

    rd = min(NH, N - p * NH)

    @pl.when(s == 0)
    def _():
      pltpu.sync_copy(acc.at[pl.ds(0, rd)],
                      out.at[pl.ds(c * N + lo, rd)])

    plsc.subcore_barrier()


R = 1000
GRID = N // R


def _tc_pre_body(x_ref, w1_ref, degp_ref, xs_ref, dinvr_ref):
  deg = 1.0 + degp_ref[:, 0:1]
  dinv = lax.rsqrt(deg)
  xw = jnp.dot(x_ref[...], w1_ref[...],
               preferred_element_type=jnp.float32)
  xs = xw * dinv
  xs_ref[0] = xs[:, :HH]
  xs_ref[1] = xs[:, HH:]
  dinvr_ref[...] = jnp.broadcast_to(dinv, (R, HH))


def _tc_pre(x, w1, degp):
  return pl.pallas_call(
      _tc_pre_body,
      grid=(GRID,),
      in_specs=[
          pl.BlockSpec((R, F_IN), lambda i: (i, 0)),
          pl.BlockSpec((F_IN, H), lambda i: (0, 0)),
          pl.BlockSpec((R, HH), lambda i: (i, 0)),
      ],
      out_specs=[
          pl.BlockSpec((2, R, HH), lambda i: (0, i, 0)),
          pl.BlockSpec((R, HH), lambda i: (i, 0)),
      ],
      out_shape=[
          jax.ShapeDtypeStruct((2, N, HH), jnp.float32),
          jax.ShapeDtypeStruct((N, HH), jnp.float32),
      ],
  )(x, w1, degp)


def _tc_post_a_body(agg_ref, xs_ref, dinvr_ref, b_ref, a_ref, st_ref):
  i = pl.program_id(0)
  dv = dinvr_ref[:, 0:1]
  left = (agg_ref[0] + xs_ref[0]) * dv
  right = (agg_ref[1] + xs_ref[1]) * dv
  a = jnp.concatenate([left, right], axis=1) + b_ref[...]
  a = jnp.maximum(a, 0.0)
  a_ref[...] = a
  s1 = jnp.sum(a, axis=0, keepdims=True)
  s2 = jnp.sum(a * a, axis=0, keepdims=True)

  @pl.when(i == 0)
  def _():
    st_ref[...] = jnp.zeros((8, H), jnp.float32)

  st_ref[...] += jnp.concatenate(
      [s1, s2, jnp.zeros((6, H), jnp.float32)], axis=0)


def _tc_post_a(agg, xs, dinvr, b):
  return pl.pallas_call(
      _tc_post_a_body,
      grid=(GRID,),
      in_specs=[
          pl.BlockSpec((2, R, HH), lambda i: (0, i, 0)),
          pl.BlockSpec((2, R, HH), lambda i: (0, i, 0)),
          pl.BlockSpec((R, HH), lambda i: (i, 0)),
          pl.BlockSpec((1, H), lambda i: (0, 0)),
      ],
      out_specs=[
          pl.BlockSpec((R, H), lambda i: (i, 0)),
          pl.BlockSpec((8, H), lambda i: (0, 0)),
      ],
      out_shape=[
          jax.ShapeDtypeStruct((N, H), jnp.float32),
          jax.ShapeDtypeStruct((8, H), jnp.float32),
      ],
  )(agg, xs, dinvr, b)


def _bn_from_stats(a, st_ref, g_ref, be_ref):
  m = st_ref[0:1, :] * (1.0 / N)
  ex2 = st_ref[1:2, :] * (1.0 / N)
  var = ex2 - m * m
  return (a - m) * lax.rsqrt(var + 1e-5) * g_ref[...] + be_ref[...]


def _tc_post_b_body(a_ref, st_ref, g_ref, be_ref, w2_ref, dinvr_ref,
                    h1_ref, xs2_ref):
  h1 = _bn_from_stats(a_ref[...], st_ref, g_ref, be_ref)
  h1_ref[...] = h1
  h1w = jnp.dot(h1, w2_ref[...], preferred_element_type=jnp.float32)
  xs2 = h1w * dinvr_ref[:, 0:1]
  xs2_ref[0] = xs2[:, :HH]
  xs2_ref[1] = xs2[:, HH:]


def _tc_post_b(a, st, g, be, w2, dinvr):
  return pl.pallas_call(
      _tc_post_b_body,
      grid=(GRID,),
      in_specs=[
          pl.BlockSpec((R, H), lambda i: (i, 0)),
          pl.BlockSpec((8, H), lambda i: (0, 0)),
          pl.BlockSpec((1, H), lambda i: (0, 0)),
          pl.BlockSpec((1, H), lambda i: (0, 0)),
          pl.BlockSpec((H, H), lambda i: (0, 0)),
          pl.BlockSpec((R, HH), lambda i: (i, 0)),
      ],
      out_specs=[
          pl.BlockSpec((R, H), lambda i: (i, 0)),
          pl.BlockSpec((2, R, HH), lambda i: (0, i, 0)),
      ],
      out_shape=[
          jax.ShapeDtypeStruct((N, H), jnp.float32),
          jax.ShapeDtypeStruct((2, N, HH), jnp.float32),
      ],
  )(a, st, g, be, w2, dinvr)


def _tc_final_body(a2_ref, st_ref, g_ref, be_ref, h1_ref, x_ref,
                   wih1t_ref, bs1_ref, wih2t_ref, bs2_ref, wlt_ref, bl_ref,
                   out_ref):
  h2 = _bn_from_stats(a2_ref[...], st_ref, g_ref, be_ref)
  xc = jnp.concatenate([h1_ref[...], h2], axis=1)
  gates = jnp.dot(xc, wih1t_ref[...],
                  preferred_element_type=jnp.float32) + bs1_ref[...]
  ig = jax.nn.sigmoid(gates[:, :H])
  gg = jnp.tanh(gates[:, 2 * H:3 * H])
  og = jax.nn.sigmoid(gates[:, 3 * H:])
  hn1 = og * jnp.tanh(ig * gg)
  gates2 = jnp.dot(hn1, wih2t_ref[...],
                   preferred_element_type=jnp.float32) + bs2_ref[...]
  ig2 = jax.nn.sigmoid(gates2[:, :H])
  gg2 = jnp.tanh(gates2[:, 2 * H:3 * H])
  og2 = jax.nn.sigmoid(gates2[:, 3 * H:])
  hn2 = og2 * jnp.tanh(ig2 * gg2)
  hcat = jnp.concatenate([hn1, hn2, x_ref[...]], axis=1)
  hcat = jnp.maximum(hcat, 0.0)
  out_ref[...] = jnp.dot(hcat, wlt_ref[...],
                         preferred_element_type=jnp.float32) + bl_ref[...]


def _tc_final(a2, st, g, be, h1, x, wih1t, bs1, wih2t, bs2, wlt, bl):
  din = 2 * H + F_IN
  return pl.pallas_call(
      _tc_final_body,
      grid=(GRID,),
      in_specs=[
          pl.BlockSpec((R, H), lambda i: (i, 0)),
          pl.BlockSpec((8, H), lambda i: (0, 0)),
          pl.BlockSpec((1, H), lambda i: (0, 0)),
          pl.BlockSpec((1, H), lambda i: (0, 0)),
          pl.BlockSpec((R, H), lambda i: (i, 0)),
          pl.BlockSpec((R, F_IN), lambda i: (i, 0)),
          pl.BlockSpec((2 * H, 4 * H), lambda i: (0, 0)),
          pl.BlockSpec((1, 4 * H), lambda i: (0, 0)),
          pl.BlockSpec((H, 4 * H), lambda i: (0, 0)),
          pl.BlockSpec((1, 4 * H), lambda i: (0, 0)),
          pl.BlockSpec((din, T_OUT), lambda i: (0, 0)),
          pl.BlockSpec((1, T_OUT), lambda i: (0, 0)),
      ],
      out_specs=pl.BlockSpec((R, T_OUT), lambda i: (i, 0)),
      out_shape=jax.ShapeDtypeStruct((N, T_OUT), jnp.float32),
  )(a2, st, g, be, h1, x, wih1t, bs1, wih2t, bs2, wlt, bl)


def kernel(x, edge_index, edge_weight, W1, b1, g1, be1, W2, b2, g2, be2,
           Wih1, Whh1, bih1, bhh1, Wih2, Whh2, bih2, bhh2, Wl, bl):
  src = edge_index[0]
  dst = edge_index[1]
  ew = edge_weight

  src_agg = src.reshape(NS, NCHK, CH)
  dst_agg = dst.reshape(NS, NCHK, CH)
  ew_agg = ew.reshape(NS, EPT)

  ones8 = jnp.ones((8, HH), jnp.float32)
  src_zero = jnp.zeros((NS, NCHK, CH), jnp.int32)
  degp = _sc_agg(ones8, src_zero, dst_agg, ew_agg)

  xs1, dinvr = _tc_pre(x, W1, degp)
  agg1 = _sc_agg(xs1.reshape(2 * N, HH), src_agg, dst_agg, ew_agg)
  a1, st1 = _tc_post_a(agg1.reshape(2, N, HH), xs1, dinvr,
                       b1.reshape(1, H))
  h1, xs2 = _tc_post_b(a1, st1, g1.reshape(1, H), be1.reshape(1, H), W2,
                       dinvr)
  agg2 = _sc_agg(xs2.reshape(2 * N, HH), src_agg, dst_agg, ew_agg)
  a2, st2 = _tc_post_a(agg2.reshape(2, N, HH), xs2, dinvr,
                       b2.reshape(1, H))
  out = _tc_final(a2, st2, g2.reshape(1, H), be2.reshape(1, H), h1, x,
                  Wih1.T, (bih1 + bhh1).reshape(1, 4 * H),
                  Wih2.T, (bih2 + bhh2).reshape(1, 4 * H),
                  Wl.T, bl.reshape(1, T_OUT))
  return out


# speedup vs baseline: 6.4499x; 6.4499x over previous
import functools

import jax
import jax.numpy as jnp
from jax import lax
from jax.experimental import pallas as pl
from jax.experimental.pallas import tpu as pltpu
from jax.experimental.pallas import tpu_sc as plsc

N = 10000
F_IN = 128
H = 256
HH = 128
T_OUT = 10
E = 320000
NS = 16
NC = 2
EPT = E // NS
CH = 80
NCHK = EPT // CH
EPW = E // (NS * NC)
DNCHK = EPW // CH

NH = 1496
NPASS = 7
NHA = 1536
AZR = NHA // NS
TRASH = NH

_mesh = plsc.VectorSubcoreMesh(core_axis_name="c", subcore_axis_name="s")


CPAD = EPT + 2 * CH


@functools.partial(
    pl.kernel,
    out_type=jax.ShapeDtypeStruct((NC * N, HH), jnp.float32),
    mesh=_mesh,
    scratch_types=[
        pltpu.VMEM((EPT + 16,), jnp.int32),
        pltpu.VMEM((EPT + 16,), jnp.int32),
        pltpu.VMEM((EPT + 16,), jnp.float32),
        pltpu.VMEM((CPAD,), jnp.int32),
        pltpu.VMEM((CH, HH), jnp.float32),
        pltpu.VMEM((AZR, HH), jnp.float32),
        pltpu.VMEM((CH,), jnp.int32),
        pltpu.VMEM((CH,), jnp.int32),
        pltpu.VMEM((CH,), jnp.float32),
        pltpu.VMEM_SHARED((NHA, HH), jnp.float32),
        pltpu.SemaphoreType.DMA,
    ],
    compiler_params=pltpu.CompilerParams(needs_layout_passes=False),
)
def _sc_agg(xs_flat, src_w, dst_w, ew_w, out, rsrc, rdst, rew, ceid,
            rows, zbuf, gidx, dst2, wbuf, acc, sem):
  c = lax.axis_index("c")
  s = lax.axis_index("s")
  ebase = s * EPT
  pltpu.sync_copy(src_w.at[pl.ds(ebase, EPT)], rsrc.at[pl.ds(0, EPT)])
  pltpu.sync_copy(dst_w.at[pl.ds(ebase, EPT)], rdst.at[pl.ds(0, EPT)])
  pltpu.sync_copy(ew_w.at[pl.ds(ebase, EPT)], rew.at[pl.ds(0, EPT)])

  zeros16 = jnp.zeros((16,), jnp.float32)
  iota16 = lax.iota(jnp.int32, 16)
  rsrc[pl.ds(EPT, 16)] = jnp.zeros((16,), jnp.int32)
  rdst[pl.ds(EPT, 16)] = jnp.full((16,), -1, jnp.int32)
  rew[pl.ds(EPT, 16)] = zeros16
  pad16 = jnp.full((16,), EPT, jnp.int32)

  def zb_body(i, _):
    for f in range(HH // 16):
      zbuf[i, pl.ds(f * 16, 16)] = zeros16
    return 0

  lax.fori_loop(0, AZR, zb_body, 0, unroll=2)

  cbase = c * (xs_flat.shape[0] // 2)

  for p in range(NPASS):
    pltpu.sync_copy(zbuf, acc.at[pl.ds(s * AZR, AZR)])
    plsc.subcore_barrier()

    lo = p * NH

    def comp_body(g, pos):
      sl = pl.ds(g * 16, 16)
      d16 = rdst[sl] - lo
      ok = (d16 >= 0) & (d16 < NH)
      key = jnp.where(ok, iota16, iota16 + 16)
      _, eid = plsc.sort_key_val(key, g * 16 + iota16)
      ceid[pl.ds(pos, 16)] = eid
      cnt = plsc.all_reduce_population_count(ok)
      return pos + cnt[0]

    pos = lax.fori_loop(0, EPT // 16, comp_body, 0)

    for g in range(CH // 16 + 1):
      ceid[pl.ds(pos + g * 16, 16)] = pad16
    nchunks = (pos + CH - 1) // CH

    def chunk_body(k, _):
      for m in range(CH // 16):
        sl = pl.ds(m * 16, 16)
        eid16 = ceid[pl.ds(k * CH + m * 16, 16)]
        d16 = plsc.load_gather(rdst, [eid16]) - lo
        dok = (d16 >= 0) & (d16 < NH)
        dst2[sl] = jnp.where(dok, d16, TRASH)
        gidx[sl] = plsc.load_gather(rsrc, [eid16]) + cbase
        wbuf[sl] = plsc.load_gather(rew, [eid16])
      pltpu.async_copy(xs_flat.at[gidx], rows, sem).wait()

      def edge_body(m, _):
        w16 = wbuf[pl.ds(m * 16, 16)]
        for j in range(16):
          e = m * 16 + j
          w = w16[j]
          for f in range(HH // 16):
            sl = pl.ds(f * 16, 16)
            rows[e, sl] = rows[e, sl] * w
        return 0

      lax.fori_loop(0, CH // 16, edge_body, 0)
      pltpu.sync_copy(rows, acc.at[dst2], add=True)
      return 0

    lax.fori_loop(0, nchunks, chunk_body, 0)
    plsc.subcore_barrier()

    rd = min(NH, N - p * NH)

    @pl.when(s == 0)
    def _():
      pltpu.sync_copy(acc.at[pl.ds(0, rd)],
                      out.at[pl.ds(c * N + lo, rd)])

    plsc.subcore_barrier()


R = 1000
GRID = N // R


def _tc_pre_body(x_ref, w1_ref, degp_ref, xs_ref, dinvr_ref):
  deg = 1.0 + degp_ref[:, 0:1]
  dinv = lax.rsqrt(deg)
  xw = jnp.dot(x_ref[...], w1_ref[...],
               preferred_element_type=jnp.float32)
  xs = xw * dinv
  xs_ref[0] = xs[:, :HH]
  xs_ref[1] = xs[:, HH:]
  dinvr_ref[...] = jnp.broadcast_to(dinv, (R, HH))


def _tc_pre(x, w1, degp):
  return pl.pallas_call(
      _tc_pre_body,
      grid=(GRID,),
      in_specs=[
          pl.BlockSpec((R, F_IN), lambda i: (i, 0)),
          pl.BlockSpec((F_IN, H), lambda i: (0, 0)),
          pl.BlockSpec((R, HH), lambda i: (i, 0)),
      ],
      out_specs=[
          pl.BlockSpec((2, R, HH), lambda i: (0, i, 0)),
          pl.BlockSpec((R, HH), lambda i: (i, 0)),
      ],
      out_shape=[
          jax.ShapeDtypeStruct((2, N, HH), jnp.float32),
          jax.ShapeDtypeStruct((N, HH), jnp.float32),
      ],
  )(x, w1, degp)


def _tc_post_a_body(agg_ref, xs_ref, dinvr_ref, b_ref, a_ref, st_ref):
  i = pl.program_id(0)
  dv = dinvr_ref[:, 0:1]
  left = (agg_ref[0] + xs_ref[0]) * dv
  right = (agg_ref[1] + xs_ref[1]) * dv
  a = jnp.concatenate([left, right], axis=1) + b_ref[...]
  a = jnp.maximum(a, 0.0)
  a_ref[...] = a
  s1 = jnp.sum(a, axis=0, keepdims=True)
  s2 = jnp.sum(a * a, axis=0, keepdims=True)

  @pl.when(i == 0)
  def _():
    st_ref[...] = jnp.zeros((8, H), jnp.float32)

  st_ref[...] += jnp.concatenate(
      [s1, s2, jnp.zeros((6, H), jnp.float32)], axis=0)


def _tc_post_a(agg, xs, dinvr, b):
  return pl.pallas_call(
      _tc_post_a_body,
      grid=(GRID,),
      in_specs=[
          pl.BlockSpec((2, R, HH), lambda i: (0, i, 0)),
          pl.BlockSpec((2, R, HH), lambda i: (0, i, 0)),
          pl.BlockSpec((R, HH), lambda i: (i, 0)),
          pl.BlockSpec((1, H), lambda i: (0, 0)),
      ],
      out_specs=[
          pl.BlockSpec((R, H), lambda i: (i, 0)),
          pl.BlockSpec((8, H), lambda i: (0, 0)),
      ],
      out_shape=[
          jax.ShapeDtypeStruct((N, H), jnp.float32),
          jax.ShapeDtypeStruct((8, H), jnp.float32),
      ],
  )(agg, xs, dinvr, b)


def _bn_from_stats(a, st_ref, g_ref, be_ref):
  m = st_ref[0:1, :] * (1.0 / N)
  ex2 = st_ref[1:2, :] * (1.0 / N)
  var = ex2 - m * m
  return (a - m) * lax.rsqrt(var + 1e-5) * g_ref[...] + be_ref[...]


def _tc_post_b_body(a_ref, st_ref, g_ref, be_ref, w2_ref, dinvr_ref,
                    h1_ref, xs2_ref):
  h1 = _bn_from_stats(a_ref[...], st_ref, g_ref, be_ref)
  h1_ref[...] = h1
  h1w = jnp.dot(h1, w2_ref[...], preferred_element_type=jnp.float32)
  xs2 = h1w * dinvr_ref[:, 0:1]
  xs2_ref[0] = xs2[:, :HH]
  xs2_ref[1] = xs2[:, HH:]


def _tc_post_b(a, st, g, be, w2, dinvr):
  return pl.pallas_call(
      _tc_post_b_body,
      grid=(GRID,),
      in_specs=[
          pl.BlockSpec((R, H), lambda i: (i, 0)),
          pl.BlockSpec((8, H), lambda i: (0, 0)),
          pl.BlockSpec((1, H), lambda i: (0, 0)),
          pl.BlockSpec((1, H), lambda i: (0, 0)),
          pl.BlockSpec((H, H), lambda i: (0, 0)),
          pl.BlockSpec((R, HH), lambda i: (i, 0)),
      ],
      out_specs=[
          pl.BlockSpec((R, H), lambda i: (i, 0)),
          pl.BlockSpec((2, R, HH), lambda i: (0, i, 0)),
      ],
      out_shape=[
          jax.ShapeDtypeStruct((N, H), jnp.float32),
          jax.ShapeDtypeStruct((2, N, HH), jnp.float32),
      ],
  )(a, st, g, be, w2, dinvr)


def _tc_final_body(a2_ref, st_ref, g_ref, be_ref, h1_ref, x_ref,
                   wih1t_ref, bs1_ref, wih2t_ref, bs2_ref, wlt_ref, bl_ref,
                   out_ref):
  h2 = _bn_from_stats(a2_ref[...], st_ref, g_ref, be_ref)
  xc = jnp.concatenate([h1_ref[...], h2], axis=1)
  gates = jnp.dot(xc, wih1t_ref[...],
                  preferred_element_type=jnp.float32) + bs1_ref[...]
  ig = jax.nn.sigmoid(gates[:, :H])
  gg = jnp.tanh(gates[:, 2 * H:3 * H])
  og = jax.nn.sigmoid(gates[:, 3 * H:])
  hn1 = og * jnp.tanh(ig * gg)
  gates2 = jnp.dot(hn1, wih2t_ref[...],
                   preferred_element_type=jnp.float32) + bs2_ref[...]
  ig2 = jax.nn.sigmoid(gates2[:, :H])
  gg2 = jnp.tanh(gates2[:, 2 * H:3 * H])
  og2 = jax.nn.sigmoid(gates2[:, 3 * H:])
  hn2 = og2 * jnp.tanh(ig2 * gg2)
  hcat = jnp.concatenate([hn1, hn2, x_ref[...]], axis=1)
  hcat = jnp.maximum(hcat, 0.0)
  out_ref[...] = jnp.dot(hcat, wlt_ref[...],
                         preferred_element_type=jnp.float32) + bl_ref[...]


def _tc_final(a2, st, g, be, h1, x, wih1t, bs1, wih2t, bs2, wlt, bl):
  din = 2 * H + F_IN
  return pl.pallas_call(
      _tc_final_body,
      grid=(GRID,),
      in_specs=[
          pl.BlockSpec((R, H), lambda i: (i, 0)),
          pl.BlockSpec((8, H), lambda i: (0, 0)),
          pl.BlockSpec((1, H), lambda i: (0, 0)),
          pl.BlockSpec((1, H), lambda i: (0, 0)),
          pl.BlockSpec((R, H), lambda i: (i, 0)),
          pl.BlockSpec((R, F_IN), lambda i: (i, 0)),
          pl.BlockSpec((2 * H, 4 * H), lambda i: (0, 0)),
          pl.BlockSpec((1, 4 * H), lambda i: (0, 0)),
          pl.BlockSpec((H, 4 * H), lambda i: (0, 0)),
          pl.BlockSpec((1, 4 * H), lambda i: (0, 0)),
          pl.BlockSpec((din, T_OUT), lambda i: (0, 0)),
          pl.BlockSpec((1, T_OUT), lambda i: (0, 0)),
      ],
      out_specs=pl.BlockSpec((R, T_OUT), lambda i: (i, 0)),
      out_shape=jax.ShapeDtypeStruct((N, T_OUT), jnp.float32),
  )(a2, st, g, be, h1, x, wih1t, bs1, wih2t, bs2, wlt, bl)


def kernel(x, edge_index, edge_weight, W1, b1, g1, be1, W2, b2, g2, be2,
           Wih1, Whh1, bih1, bhh1, Wih2, Whh2, bih2, bhh2, Wl, bl):
  src = edge_index[0]
  dst = edge_index[1]
  ew = edge_weight

  ones8 = jnp.ones((8, HH), jnp.float32)
  src_zero = jnp.zeros((E,), jnp.int32)
  degp = _sc_agg(ones8, src_zero, dst, ew)

  xs1, dinvr = _tc_pre(x, W1, degp)
  agg1 = _sc_agg(xs1.reshape(2 * N, HH), src, dst, ew)
  a1, st1 = _tc_post_a(agg1.reshape(2, N, HH), xs1, dinvr,
                       b1.reshape(1, H))
  h1, xs2 = _tc_post_b(a1, st1, g1.reshape(1, H), be1.reshape(1, H), W2,
                       dinvr)
  agg2 = _sc_agg(xs2.reshape(2 * N, HH), src, dst, ew)
  a2, st2 = _tc_post_a(agg2.reshape(2, N, HH), xs2, dinvr,
                       b2.reshape(1, H))
  out = _tc_final(a2, st2, g2.reshape(1, H), be2.reshape(1, H), h1, x,
                  Wih1.T, (bih1 + bhh1).reshape(1, 4 * H),
                  Wih2.T, (bih2 + bhh2).reshape(1, 4 * H),
                  Wl.T, bl.reshape(1, T_OUT))
  return out


# speedup vs baseline: 6.5464x; 1.0150x over previous
import functools

import jax
import jax.numpy as jnp
from jax import lax
from jax.experimental import pallas as pl
from jax.experimental.pallas import tpu as pltpu
from jax.experimental.pallas import tpu_sc as plsc

N = 10000
F_IN = 128
H = 256
HH = 128
T_OUT = 10
E = 320000
NS = 16
NC = 2
EPT = E // NS
CH = 64
NCHK = EPT // CH
EPW = E // (NS * NC)
DNCHK = EPW // CH

NH = 1016
NPASS = 10
NHA = 1024
AZR = NHA // NS
TRASH = NH

_mesh = plsc.VectorSubcoreMesh(core_axis_name="c", subcore_axis_name="s")


CPAD = EPT + 2 * CH
NBUF = 4


@functools.partial(
    pl.kernel,
    out_type=jax.ShapeDtypeStruct((NC * N, HH), jnp.float32),
    mesh=_mesh,
    scratch_types=[
        pltpu.VMEM((EPT + 16,), jnp.int32),
        pltpu.VMEM((EPT + 16,), jnp.int32),
        pltpu.VMEM((EPT + 16,), jnp.float32),
        pltpu.VMEM((CPAD,), jnp.int32),
        pltpu.VMEM((NBUF * CH, HH), jnp.float32),
        pltpu.VMEM((NBUF, CH), jnp.int32),
        pltpu.VMEM((NBUF, CH), jnp.int32),
        pltpu.VMEM((NBUF, CH), jnp.float32),
        pltpu.VMEM_SHARED((NHA, HH), jnp.float32),
        pltpu.SemaphoreType.DMA,
        pltpu.SemaphoreType.DMA,
        pltpu.SemaphoreType.DMA,
        pltpu.SemaphoreType.DMA,
        pltpu.SemaphoreType.DMA,
        pltpu.SemaphoreType.DMA,
        pltpu.SemaphoreType.DMA,
        pltpu.SemaphoreType.DMA,
    ],
    compiler_params=pltpu.CompilerParams(needs_layout_passes=False),
)
def _sc_agg(xs_flat, src_w, dst_w, ew_w, out, rsrc, rdst, rew, ceid,
            rows, gidx, dst2, wbuf, acc, g0, g1, g2, g3, s0, s1, s2, s3):
  gsems = [g0, g1, g2, g3]
  ssems = [s0, s1, s2, s3]
  c = lax.axis_index("c")
  s = lax.axis_index("s")
  ebase = s * EPT
  pltpu.sync_copy(src_w.at[pl.ds(ebase, EPT)], rsrc.at[pl.ds(0, EPT)])
  pltpu.sync_copy(dst_w.at[pl.ds(ebase, EPT)], rdst.at[pl.ds(0, EPT)])
  pltpu.sync_copy(ew_w.at[pl.ds(ebase, EPT)], rew.at[pl.ds(0, EPT)])

  zeros16 = jnp.zeros((16,), jnp.float32)
  iota16 = lax.iota(jnp.int32, 16)
  rsrc[pl.ds(EPT, 16)] = jnp.zeros((16,), jnp.int32)
  rdst[pl.ds(EPT, 16)] = jnp.full((16,), -1, jnp.int32)
  rew[pl.ds(EPT, 16)] = zeros16
  pad16 = jnp.full((16,), EPT, jnp.int32)

  cbase = c * (xs_flat.shape[0] // 2)

  def _prep(b, k, lo):
    for m in range(CH // 16):
      sl = pl.ds(m * 16, 16)
      eid16 = ceid[pl.ds(k * CH + m * 16, 16)]
      d16 = plsc.load_gather(rdst, [eid16]) - lo
      dok = (d16 >= 0) & (d16 < NH)
      dst2[b, sl] = jnp.where(dok, d16, TRASH)
      gidx[b, sl] = plsc.load_gather(rsrc, [eid16]) + cbase
      wbuf[b, sl] = plsc.load_gather(rew, [eid16])

  def _rows(b):
    return rows.at[pl.ds(b * CH, CH)]

  def _issue_g(b):
    pltpu.async_copy(xs_flat.at[gidx.at[b]], _rows(b), gsems[b])

  def _wait_g(b):
    pltpu.make_async_copy(xs_flat.at[gidx.at[b]], _rows(b), gsems[b]).wait()

  def _issue_s(b):
    pltpu.async_copy(_rows(b), acc.at[dst2.at[b]], ssems[b], add=True)

  def _wait_s(b):
    pltpu.make_async_copy(_rows(b), acc.at[dst2.at[b]], ssems[b]).wait()

  def _compute(b):
    def edge_body(m, _):
      w16 = wbuf[b, pl.ds(m * 16, 16)]
      for j in range(16):
        e = b * CH + m * 16 + j
        w = w16[j]
        for f in range(HH // 16):
          sl = pl.ds(f * 16, 16)
          rows[e, sl] = rows[e, sl] * w
      return 0

    lax.fori_loop(0, CH // 16, edge_body, 0)

  def pass_body(p, _):
    lo = p * NH

    def zb_body(i, _):
      for f in range(HH // 16):
        rows[i, pl.ds(f * 16, 16)] = zeros16
      return 0

    lax.fori_loop(0, AZR, zb_body, 0, unroll=2)
    pltpu.sync_copy(rows.at[pl.ds(0, AZR)], acc.at[pl.ds(s * AZR, AZR)])
    plsc.subcore_barrier()

    def comp_body(g, pos):
      sl = pl.ds(g * 16, 16)
      d16 = rdst[sl] - lo
      ok = (d16 >= 0) & (d16 < NH)
      key = jnp.where(ok, iota16, iota16 + 16)
      _, eid = plsc.sort_key_val(key, g * 16 + iota16)
      ceid[pl.ds(pos, 16)] = eid
      cnt = plsc.all_reduce_population_count(ok)
      return pos + cnt[0]

    pos = lax.fori_loop(0, EPT // 16, comp_body, 0)

    for g in range(CH // 16 + 1):
      ceid[pl.ds(pos + g * 16, 16)] = pad16
    nchunks = (pos + CH - 1) // CH

    for b in range(NBUF):
      @pl.when(b < nchunks)
      def _(b=b):
        _prep(b, b, lo)
        _issue_g(b)

    def slot_body(kk, _):
      for b in range(NBUF):
        k = kk * NBUF + b

        @pl.when(k < nchunks)
        def _(b=b, k=k):
          _wait_g(b)
          _compute(b)
          _issue_s(b)

        bn = (b + 2) % NBUF
        kn = k + 2

        @pl.when((kn >= NBUF) & (kn < nchunks))
        def _(b=bn, k=kn):
          _wait_s(b)
          _prep(b, k, lo)
          _issue_g(b)

      return 0

    lax.fori_loop(0, (nchunks + NBUF - 1) // NBUF, slot_body, 0)

    for b in range(NBUF):
      @pl.when(b < nchunks)
      def _(b=b):
        _wait_s(b)

    plsc.subcore_barrier()

    @pl.when(s == 0)
    def _():
      @pl.when(p < NPASS - 1)
      def _():
        pltpu.sync_copy(acc.at[pl.ds(0, NH)],
                        out.at[pl.ds(c * N + lo, NH)])

      @pl.when(p == NPASS - 1)
      def _():
        pltpu.sync_copy(acc.at[pl.ds(0, N - (NPASS - 1) * NH)],
                        out.at[pl.ds(c * N + lo, N - (NPASS - 1) * NH)])

    plsc.subcore_barrier()
    return 0

  lax.fori_loop(0, NPASS, pass_body, 0)


R = 1000
GRID = N // R


def _tc_pre_body(x_ref, w1_ref, degp_ref, xs_ref, dinvr_ref):
  deg = 1.0 + degp_ref[:, 0:1]
  dinv = lax.rsqrt(deg)
  xw = jnp.dot(x_ref[...], w1_ref[...],
               preferred_element_type=jnp.float32)
  xs = xw * dinv
  xs_ref[0] = xs[:, :HH]
  xs_ref[1] = xs[:, HH:]
  dinvr_ref[...] = jnp.broadcast_to(dinv, (R, HH))


def _tc_pre(x, w1, degp):
  return pl.pallas_call(
      _tc_pre_body,
      grid=(GRID,),
      in_specs=[
          pl.BlockSpec((R, F_IN), lambda i: (i, 0)),
          pl.BlockSpec((F_IN, H), lambda i: (0, 0)),
          pl.BlockSpec((R, HH), lambda i: (i, 0)),
      ],
      out_specs=[
          pl.BlockSpec((2, R, HH), lambda i: (0, i, 0)),
          pl.BlockSpec((R, HH), lambda i: (i, 0)),
      ],
      out_shape=[
          jax.ShapeDtypeStruct((2, N, HH), jnp.float32),
          jax.ShapeDtypeStruct((N, HH), jnp.float32),
      ],
  )(x, w1, degp)


def _tc_post_a_body(agg_ref, xs_ref, dinvr_ref, b_ref, a_ref, st_ref):
  i = pl.program_id(0)
  dv = dinvr_ref[:, 0:1]
  left = (agg_ref[0] + xs_ref[0]) * dv
  right = (agg_ref[1] + xs_ref[1]) * dv
  a = jnp.concatenate([left, right], axis=1) + b_ref[...]
  a = jnp.maximum(a, 0.0)
  a_ref[...] = a
  s1 = jnp.sum(a, axis=0, keepdims=True)
  s2 = jnp.sum(a * a, axis=0, keepdims=True)

  @pl.when(i == 0)
  def _():
    st_ref[...] = jnp.zeros((8, H), jnp.float32)

  st_ref[...] += jnp.concatenate(
      [s1, s2, jnp.zeros((6, H), jnp.float32)], axis=0)


def _tc_post_a(agg, xs, dinvr, b):
  return pl.pallas_call(
      _tc_post_a_body,
      grid=(GRID,),
      in_specs=[
          pl.BlockSpec((2, R, HH), lambda i: (0, i, 0)),
          pl.BlockSpec((2, R, HH), lambda i: (0, i, 0)),
          pl.BlockSpec((R, HH), lambda i: (i, 0)),
          pl.BlockSpec((1, H), lambda i: (0, 0)),
      ],
      out_specs=[
          pl.BlockSpec((R, H), lambda i: (i, 0)),
          pl.BlockSpec((8, H), lambda i: (0, 0)),
      ],
      out_shape=[
          jax.ShapeDtypeStruct((N, H), jnp.float32),
          jax.ShapeDtypeStruct((8, H), jnp.float32),
      ],
  )(agg, xs, dinvr, b)


def _bn_from_stats(a, st_ref, g_ref, be_ref):
  m = st_ref[0:1, :] * (1.0 / N)
  ex2 = st_ref[1:2, :] * (1.0 / N)
  var = ex2 - m * m
  return (a - m) * lax.rsqrt(var + 1e-5) * g_ref[...] + be_ref[...]


def _tc_post_b_body(a_ref, st_ref, g_ref, be_ref, w2_ref, dinvr_ref,
                    h1_ref, xs2_ref):
  h1 = _bn_from_stats(a_ref[...], st_ref, g_ref, be_ref)
  h1_ref[...] = h1
  h1w = jnp.dot(h1, w2_ref[...], preferred_element_type=jnp.float32)
  xs2 = h1w * dinvr_ref[:, 0:1]
  xs2_ref[0] = xs2[:, :HH]
  xs2_ref[1] = xs2[:, HH:]


def _tc_post_b(a, st, g, be, w2, dinvr):
  return pl.pallas_call(
      _tc_post_b_body,
      grid=(GRID,),
      in_specs=[
          pl.BlockSpec((R, H), lambda i: (i, 0)),
          pl.BlockSpec((8, H), lambda i: (0, 0)),
          pl.BlockSpec((1, H), lambda i: (0, 0)),
          pl.BlockSpec((1, H), lambda i: (0, 0)),
          pl.BlockSpec((H, H), lambda i: (0, 0)),
          pl.BlockSpec((R, HH), lambda i: (i, 0)),
      ],
      out_specs=[
          pl.BlockSpec((R, H), lambda i: (i, 0)),
          pl.BlockSpec((2, R, HH), lambda i: (0, i, 0)),
      ],
      out_shape=[
          jax.ShapeDtypeStruct((N, H), jnp.float32),
          jax.ShapeDtypeStruct((2, N, HH), jnp.float32),
      ],
  )(a, st, g, be, w2, dinvr)


def _tc_final_body(a2_ref, st_ref, g_ref, be_ref, h1_ref, x_ref,
                   wih1t_ref, bs1_ref, wih2t_ref, bs2_ref, wlt_ref, bl_ref,
                   out_ref):
  h2 = _bn_from_stats(a2_ref[...], st_ref, g_ref, be_ref)
  xc = jnp.concatenate([h1_ref[...], h2], axis=1)
  gates = jnp.dot(xc, wih1t_ref[...],
                  preferred_element_type=jnp.float32) + bs1_ref[...]
  ig = jax.nn.sigmoid(gates[:, :H])
  gg = jnp.tanh(gates[:, 2 * H:3 * H])
  og = jax.nn.sigmoid(gates[:, 3 * H:])
  hn1 = og * jnp.tanh(ig * gg)
  gates2 = jnp.dot(hn1, wih2t_ref[...],
                   preferred_element_type=jnp.float32) + bs2_ref[...]
  ig2 = jax.nn.sigmoid(gates2[:, :H])
  gg2 = jnp.tanh(gates2[:, 2 * H:3 * H])
  og2 = jax.nn.sigmoid(gates2[:, 3 * H:])
  hn2 = og2 * jnp.tanh(ig2 * gg2)
  hcat = jnp.concatenate([hn1, hn2, x_ref[...]], axis=1)
  hcat = jnp.maximum(hcat, 0.0)
  out_ref[...] = jnp.dot(hcat, wlt_ref[...],
                         preferred_element_type=jnp.float32) + bl_ref[...]


def _tc_final(a2, st, g, be, h1, x, wih1t, bs1, wih2t, bs2, wlt, bl):
  din = 2 * H + F_IN
  return pl.pallas_call(
      _tc_final_body,
      grid=(GRID,),
      in_specs=[
          pl.BlockSpec((R, H), lambda i: (i, 0)),
          pl.BlockSpec((8, H), lambda i: (0, 0)),
          pl.BlockSpec((1, H), lambda i: (0, 0)),
          pl.BlockSpec((1, H), lambda i: (0, 0)),
          pl.BlockSpec((R, H), lambda i: (i, 0)),
          pl.BlockSpec((R, F_IN), lambda i: (i, 0)),
          pl.BlockSpec((2 * H, 4 * H), lambda i: (0, 0)),
          pl.BlockSpec((1, 4 * H), lambda i: (0, 0)),
          pl.BlockSpec((H, 4 * H), lambda i: (0, 0)),
          pl.BlockSpec((1, 4 * H), lambda i: (0, 0)),
          pl.BlockSpec((din, T_OUT), lambda i: (0, 0)),
          pl.BlockSpec((1, T_OUT), lambda i: (0, 0)),
      ],
      out_specs=pl.BlockSpec((R, T_OUT), lambda i: (i, 0)),
      out_shape=jax.ShapeDtypeStruct((N, T_OUT), jnp.float32),
  )(a2, st, g, be, h1, x, wih1t, bs1, wih2t, bs2, wlt, bl)


def kernel(x, edge_index, edge_weight, W1, b1, g1, be1, W2, b2, g2, be2,
           Wih1, Whh1, bih1, bhh1, Wih2, Whh2, bih2, bhh2, Wl, bl):
  src = edge_index[0]
  dst = edge_index[1]
  ew = edge_weight

  ones8 = jnp.ones((8, HH), jnp.float32)
  src_zero = jnp.zeros((E,), jnp.int32)
  degp = _sc_agg(ones8, src_zero, dst, ew)

  xs1, dinvr = _tc_pre(x, W1, degp)
  agg1 = _sc_agg(xs1.reshape(2 * N, HH), src, dst, ew)
  a1, st1 = _tc_post_a(agg1.reshape(2, N, HH), xs1, dinvr,
                       b1.reshape(1, H))
  h1, xs2 = _tc_post_b(a1, st1, g1.reshape(1, H), be1.reshape(1, H), W2,
                       dinvr)
  agg2 = _sc_agg(xs2.reshape(2 * N, HH), src, dst, ew)
  a2, st2 = _tc_post_a(agg2.reshape(2, N, HH), xs2, dinvr,
                       b2.reshape(1, H))
  out = _tc_final(a2, st2, g2.reshape(1, H), be2.reshape(1, H), h1, x,
                  Wih1.T, (bih1 + bhh1).reshape(1, 4 * H),
                  Wih2.T, (bih2 + bhh2).reshape(1, 4 * H),
                  Wl.T, bl.reshape(1, T_OUT))
  return out


# speedup vs baseline: 44.9065x; 6.8597x over previous
import functools

import jax
import jax.numpy as jnp
from jax import lax
from jax.experimental import pallas as pl
from jax.experimental.pallas import tpu as pltpu
from jax.experimental.pallas import tpu_sc as plsc

N = 10000
F_IN = 128
H = 256
HH = 128
T_OUT = 10
E = 320000
NS = 16
NC = 2
EPT = E // NS
CH = 64
NCHK = EPT // CH
EPW = E // (NS * NC)
DNCHK = EPW // CH

NH = 1016
NPASS = 10
NHA = 1024
AZR = NHA // NS
TRASH = NH

_mesh = plsc.VectorSubcoreMesh(core_axis_name="c", subcore_axis_name="s")


CPAD = EPT + 2 * CH
NBUF = 4


@functools.partial(
    pl.kernel,
    out_type=jax.ShapeDtypeStruct((NC * N, HH), jnp.float32),
    mesh=_mesh,
    scratch_types=[
        pltpu.VMEM((EPT + 16,), jnp.int32),
        pltpu.VMEM((EPT + 16,), jnp.int32),
        pltpu.VMEM((EPT + 16,), jnp.float32),
        pltpu.VMEM((CPAD,), jnp.int32),
        pltpu.VMEM((NBUF * CH, HH), jnp.float32),
        pltpu.VMEM((NBUF, CH), jnp.int32),
        pltpu.VMEM((NBUF, CH), jnp.int32),
        pltpu.VMEM((NBUF, CH), jnp.float32),
        pltpu.VMEM_SHARED((NHA, HH), jnp.float32),
        pltpu.SemaphoreType.DMA,
        pltpu.SemaphoreType.DMA,
        pltpu.SemaphoreType.DMA,
        pltpu.SemaphoreType.DMA,
        pltpu.SemaphoreType.DMA,
        pltpu.SemaphoreType.DMA,
        pltpu.SemaphoreType.DMA,
        pltpu.SemaphoreType.DMA,
    ],
    compiler_params=pltpu.CompilerParams(needs_layout_passes=False),
)
def _sc_agg(xs_flat, src_w, dst_w, ew_w, out, rsrc, rdst, rew, ceid,
            rows, gidx, dst2, wbuf, acc, g0, g1, g2, g3, s0, s1, s2, s3):
  gsems = [g0, g1, g2, g3]
  ssems = [s0, s1, s2, s3]
  c = lax.axis_index("c")
  s = lax.axis_index("s")
  ebase = s * EPT
  pltpu.sync_copy(src_w.at[pl.ds(ebase, EPT)], rsrc.at[pl.ds(0, EPT)])
  pltpu.sync_copy(dst_w.at[pl.ds(ebase, EPT)], rdst.at[pl.ds(0, EPT)])
  pltpu.sync_copy(ew_w.at[pl.ds(ebase, EPT)], rew.at[pl.ds(0, EPT)])

  zeros16 = jnp.zeros((16,), jnp.float32)
  iota16 = lax.iota(jnp.int32, 16)
  rsrc[pl.ds(EPT, 16)] = jnp.zeros((16,), jnp.int32)
  rdst[pl.ds(EPT, 16)] = jnp.full((16,), -1, jnp.int32)
  rew[pl.ds(EPT, 16)] = zeros16
  pad16 = jnp.full((16,), EPT, jnp.int32)

  cbase = c * (xs_flat.shape[0] // 2)

  def _prep(b, k, lo):
    for m in range(CH // 16):
      sl = pl.ds(m * 16, 16)
      eid16 = ceid[pl.ds(k * CH + m * 16, 16)]
      d16 = plsc.load_gather(rdst, [eid16]) - lo
      dok = (d16 >= 0) & (d16 < NH)
      dst2[b, sl] = jnp.where(dok, d16, TRASH)
      gidx[b, sl] = plsc.load_gather(rsrc, [eid16]) + cbase
      wbuf[b, sl] = plsc.load_gather(rew, [eid16])

  def _rows(b):
    return rows.at[pl.ds(b * CH, CH)]

  def _issue_g(b):
    pltpu.async_copy(xs_flat.at[gidx.at[b]], _rows(b), gsems[b])

  def _wait_g(b):
    pltpu.make_async_copy(xs_flat.at[gidx.at[b]], _rows(b), gsems[b]).wait()

  def _issue_s(b):
    pltpu.async_copy(_rows(b), acc.at[dst2.at[b]], ssems[b], add=True)

  def _wait_s(b):
    pltpu.make_async_copy(_rows(b), acc.at[dst2.at[b]], ssems[b]).wait()

  def _compute(b):
    def edge_body(m, _):
      w16 = wbuf[b, pl.ds(m * 16, 16)]
      for j in range(16):
        e = b * CH + m * 16 + j
        w = w16[j]
        for f in range(HH // 16):
          sl = pl.ds(f * 16, 16)
          rows[e, sl] = rows[e, sl] * w
      return 0

    lax.fori_loop(0, CH // 16, edge_body, 0)

  def pass_body(p, _):
    lo = p * NH

    def zb_body(i, _):
      for f in range(HH // 16):
        rows[i, pl.ds(f * 16, 16)] = zeros16
      return 0

    lax.fori_loop(0, AZR, zb_body, 0, unroll=2)
    pltpu.sync_copy(rows.at[pl.ds(0, AZR)], acc.at[pl.ds(s * AZR, AZR)])
    plsc.subcore_barrier()

    def comp_body(g, pos):
      sl = pl.ds(g * 16, 16)
      d16 = rdst[sl] - lo
      ok = (d16 >= 0) & (d16 < NH)
      key = jnp.where(ok, iota16, iota16 + 16)
      _, eid = plsc.sort_key_val(key, g * 16 + iota16)
      ceid[pl.ds(pos, 16)] = eid
      cnt = plsc.all_reduce_population_count(ok)
      return pos + cnt[0]

    pos = lax.fori_loop(0, EPT // 16, comp_body, 0)

    for g in range(CH // 16 + 1):
      ceid[pl.ds(pos + g * 16, 16)] = pad16
    nchunks = (pos + CH - 1) // CH

    for b in range(NBUF):
      @pl.when(b < nchunks)
      def _(b=b):
        _prep(b, b, lo)
        _issue_g(b)

    def slot_body(kk, _):
      for b in range(NBUF):
        k = kk * NBUF + b

        @pl.when(k < nchunks)
        def _(b=b, k=k):
          _wait_g(b)
          _compute(b)
          _issue_s(b)

        bn = (b + 2) % NBUF
        kn = k + 2

        @pl.when((kn >= NBUF) & (kn < nchunks))
        def _(b=bn, k=kn):
          _wait_s(b)
          _prep(b, k, lo)
          _issue_g(b)

      return 0

    lax.fori_loop(0, (nchunks + NBUF - 1) // NBUF, slot_body, 0)

    for b in range(NBUF):
      @pl.when(b < nchunks)
      def _(b=b):
        _wait_s(b)

    plsc.subcore_barrier()

    @pl.when(s == 0)
    def _():
      @pl.when(p < NPASS - 1)
      def _():
        pltpu.sync_copy(acc.at[pl.ds(0, NH)],
                        out.at[pl.ds(c * N + lo, NH)])

      @pl.when(p == NPASS - 1)
      def _():
        pltpu.sync_copy(acc.at[pl.ds(0, N - (NPASS - 1) * NH)],
                        out.at[pl.ds(c * N + lo, N - (NPASS - 1) * NH)])

    plsc.subcore_barrier()
    return 0

  lax.fori_loop(0, NPASS, pass_body, 0)


R = 1000
GRID = N // R


def _tc_pre_body(x_ref, w1_ref, degp_ref, xs_ref, dinvr_ref):
  deg = 1.0 + degp_ref[:, 0:1]
  dinv = lax.rsqrt(deg)
  xw = jnp.dot(x_ref[...], w1_ref[...],
               preferred_element_type=jnp.float32)
  xs = xw * dinv
  xs_ref[0] = xs[:, :HH]
  xs_ref[1] = xs[:, HH:]
  dinvr_ref[...] = jnp.broadcast_to(dinv, (R, HH))


def _tc_pre(x, w1, degp):
  return pl.pallas_call(
      _tc_pre_body,
      grid=(GRID,),
      in_specs=[
          pl.BlockSpec((R, F_IN), lambda i: (i, 0)),
          pl.BlockSpec((F_IN, H), lambda i: (0, 0)),
          pl.BlockSpec((R, HH), lambda i: (i, 0)),
      ],
      out_specs=[
          pl.BlockSpec((2, R, HH), lambda i: (0, i, 0)),
          pl.BlockSpec((R, HH), lambda i: (i, 0)),
      ],
      out_shape=[
          jax.ShapeDtypeStruct((2, N, HH), jnp.float32),
          jax.ShapeDtypeStruct((N, HH), jnp.float32),
      ],
  )(x, w1, degp)


def _tc_post_a_body(agg_ref, xs_ref, dinvr_ref, b_ref, a_ref, st_ref):
  i = pl.program_id(0)
  dv = dinvr_ref[:, 0:1]
  left = (agg_ref[0] + xs_ref[0]) * dv
  right = (agg_ref[1] + xs_ref[1]) * dv
  a = jnp.concatenate([left, right], axis=1) + b_ref[...]
  a = jnp.maximum(a, 0.0)
  a_ref[...] = a
  s1 = jnp.sum(a, axis=0, keepdims=True)
  s2 = jnp.sum(a * a, axis=0, keepdims=True)

  @pl.when(i == 0)
  def _():
    st_ref[...] = jnp.zeros((8, H), jnp.float32)

  st_ref[...] += jnp.concatenate(
      [s1, s2, jnp.zeros((6, H), jnp.float32)], axis=0)


def _tc_post_a(agg, xs, dinvr, b):
  return pl.pallas_call(
      _tc_post_a_body,
      grid=(GRID,),
      in_specs=[
          pl.BlockSpec((2, R, HH), lambda i: (0, i, 0)),
          pl.BlockSpec((2, R, HH), lambda i: (0, i, 0)),
          pl.BlockSpec((R, HH), lambda i: (i, 0)),
          pl.BlockSpec((1, H), lambda i: (0, 0)),
      ],
      out_specs=[
          pl.BlockSpec((R, H), lambda i: (i, 0)),
          pl.BlockSpec((8, H), lambda i: (0, 0)),
      ],
      out_shape=[
          jax.ShapeDtypeStruct((N, H), jnp.float32),
          jax.ShapeDtypeStruct((8, H), jnp.float32),
      ],
  )(agg, xs, dinvr, b)


def _bn_from_stats(a, st_ref, g_ref, be_ref):
  m = st_ref[0:1, :] * (1.0 / N)
  ex2 = st_ref[1:2, :] * (1.0 / N)
  var = ex2 - m * m
  return (a - m) * lax.rsqrt(var + 1e-5) * g_ref[...] + be_ref[...]


def _tc_post_b_body(a_ref, st_ref, g_ref, be_ref, w2_ref, dinvr_ref,
                    h1_ref, xs2_ref):
  h1 = _bn_from_stats(a_ref[...], st_ref, g_ref, be_ref)
  h1_ref[...] = h1
  h1w = jnp.dot(h1, w2_ref[...], preferred_element_type=jnp.float32)
  xs2 = h1w * dinvr_ref[:, 0:1]
  xs2_ref[0] = xs2[:, :HH]
  xs2_ref[1] = xs2[:, HH:]


def _tc_post_b(a, st, g, be, w2, dinvr):
  return pl.pallas_call(
      _tc_post_b_body,
      grid=(GRID,),
      in_specs=[
          pl.BlockSpec((R, H), lambda i: (i, 0)),
          pl.BlockSpec((8, H), lambda i: (0, 0)),
          pl.BlockSpec((1, H), lambda i: (0, 0)),
          pl.BlockSpec((1, H), lambda i: (0, 0)),
          pl.BlockSpec((H, H), lambda i: (0, 0)),
          pl.BlockSpec((R, HH), lambda i: (i, 0)),
      ],
      out_specs=[
          pl.BlockSpec((R, H), lambda i: (i, 0)),
          pl.BlockSpec((2, R, HH), lambda i: (0, i, 0)),
      ],
      out_shape=[
          jax.ShapeDtypeStruct((N, H), jnp.float32),
          jax.ShapeDtypeStruct((2, N, HH), jnp.float32),
      ],
  )(a, st, g, be, w2, dinvr)


def _tc_final_body(a2_ref, st_ref, g_ref, be_ref, h1_ref, x_ref,
                   wih1t_ref, bs1_ref, wih2t_ref, bs2_ref, wlt_ref, bl_ref,
                   out_ref):
  h2 = _bn_from_stats(a2_ref[...], st_ref, g_ref, be_ref)
  xc = jnp.concatenate([h1_ref[...], h2], axis=1)
  gates = jnp.dot(xc, wih1t_ref[...],
                  preferred_element_type=jnp.float32) + bs1_ref[...]
  ig = jax.nn.sigmoid(gates[:, :H])
  gg = jnp.tanh(gates[:, 2 * H:3 * H])
  og = jax.nn.sigmoid(gates[:, 3 * H:])
  hn1 = og * jnp.tanh(ig * gg)
  gates2 = jnp.dot(hn1, wih2t_ref[...],
                   preferred_element_type=jnp.float32) + bs2_ref[...]
  ig2 = jax.nn.sigmoid(gates2[:, :H])
  gg2 = jnp.tanh(gates2[:, 2 * H:3 * H])
  og2 = jax.nn.sigmoid(gates2[:, 3 * H:])
  hn2 = og2 * jnp.tanh(ig2 * gg2)
  hcat = jnp.concatenate([hn1, hn2, x_ref[...]], axis=1)
  hcat = jnp.maximum(hcat, 0.0)
  out_ref[...] = jnp.dot(hcat, wlt_ref[...],
                         preferred_element_type=jnp.float32) + bl_ref[...]


def _tc_final(a2, st, g, be, h1, x, wih1t, bs1, wih2t, bs2, wlt, bl):
  din = 2 * H + F_IN
  return pl.pallas_call(
      _tc_final_body,
      grid=(GRID,),
      in_specs=[
          pl.BlockSpec((R, H), lambda i: (i, 0)),
          pl.BlockSpec((8, H), lambda i: (0, 0)),
          pl.BlockSpec((1, H), lambda i: (0, 0)),
          pl.BlockSpec((1, H), lambda i: (0, 0)),
          pl.BlockSpec((R, H), lambda i: (i, 0)),
          pl.BlockSpec((R, F_IN), lambda i: (i, 0)),
          pl.BlockSpec((2 * H, 4 * H), lambda i: (0, 0)),
          pl.BlockSpec((1, 4 * H), lambda i: (0, 0)),
          pl.BlockSpec((H, 4 * H), lambda i: (0, 0)),
          pl.BlockSpec((1, 4 * H), lambda i: (0, 0)),
          pl.BlockSpec((din, T_OUT), lambda i: (0, 0)),
          pl.BlockSpec((1, T_OUT), lambda i: (0, 0)),
      ],
      out_specs=pl.BlockSpec((R, T_OUT), lambda i: (i, 0)),
      out_shape=jax.ShapeDtypeStruct((N, T_OUT), jnp.float32),
  )(a2, st, g, be, h1, x, wih1t, bs1, wih2t, bs2, wlt, bl)


def kernel(x, edge_index, edge_weight, W1, b1, g1, be1, W2, b2, g2, be2,
           Wih1, Whh1, bih1, bhh1, Wih2, Whh2, bih2, bhh2, Wl, bl):
  src = edge_index[0]
  dst = edge_index[1]
  ew = edge_weight

  ones_tab = jnp.ones((2 * N, HH), jnp.float32)
  degp = _sc_agg(ones_tab, src, dst, ew)

  xs1, dinvr = _tc_pre(x, W1, degp)
  agg1 = _sc_agg(xs1.reshape(2 * N, HH), src, dst, ew)
  a1, st1 = _tc_post_a(agg1.reshape(2, N, HH), xs1, dinvr,
                       b1.reshape(1, H))
  h1, xs2 = _tc_post_b(a1, st1, g1.reshape(1, H), be1.reshape(1, H), W2,
                       dinvr)
  agg2 = _sc_agg(xs2.reshape(2 * N, HH), src, dst, ew)
  a2, st2 = _tc_post_a(agg2.reshape(2, N, HH), xs2, dinvr,
                       b2.reshape(1, H))
  out = _tc_final(a2, st2, g2.reshape(1, H), be2.reshape(1, H), h1, x,
                  Wih1.T, (bih1 + bhh1).reshape(1, 4 * H),
                  Wih2.T, (bih2 + bhh2).reshape(1, 4 * H),
                  Wl.T, bl.reshape(1, T_OUT))
  return out


# speedup vs baseline: 45.3446x; 1.0098x over previous
import functools

import jax
import jax.numpy as jnp
from jax import lax
from jax.experimental import pallas as pl
from jax.experimental.pallas import tpu as pltpu
from jax.experimental.pallas import tpu_sc as plsc

N = 10000
F_IN = 128
H = 256
HH = 128
T_OUT = 10
E = 320000
NS = 16
NC = 2
EPT = E // NS
CH = 64
NCHK = EPT // CH
EPW = E // (NS * NC)
DNCHK = EPW // CH

NH = 1016
NPASS = 10
NHA = 1024
AZR = NHA // NS
TRASH = NH

_mesh = plsc.VectorSubcoreMesh(core_axis_name="c", subcore_axis_name="s")


CPAD = EPT + 2 * CH
NBUF = 4


@functools.partial(
    pl.kernel,
    out_type=jax.ShapeDtypeStruct((NC * N, HH), jnp.float32),
    mesh=_mesh,
    scratch_types=[
        pltpu.VMEM((EPT + 16,), jnp.int32),
        pltpu.VMEM((EPT + 16,), jnp.int32),
        pltpu.VMEM((EPT + 16,), jnp.float32),
        pltpu.VMEM((CPAD,), jnp.int32),
        pltpu.VMEM((NBUF * CH, HH), jnp.float32),
        pltpu.VMEM((NBUF, CH), jnp.int32),
        pltpu.VMEM((NBUF, CH), jnp.int32),
        pltpu.VMEM((NBUF, CH), jnp.float32),
        pltpu.VMEM_SHARED((NHA, HH), jnp.float32),
        pltpu.SemaphoreType.DMA,
        pltpu.SemaphoreType.DMA,
        pltpu.SemaphoreType.DMA,
        pltpu.SemaphoreType.DMA,
        pltpu.SemaphoreType.DMA,
        pltpu.SemaphoreType.DMA,
        pltpu.SemaphoreType.DMA,
        pltpu.SemaphoreType.DMA,
    ],
    compiler_params=pltpu.CompilerParams(needs_layout_passes=False),
)
def _sc_agg(xs_flat, src_w, dst_w, ew_w, out, rsrc, rdst, rew, ceid,
            rows, gidx, dst2, wbuf, acc, g0, g1, g2, g3, s0, s1, s2, s3):
  gsems = [g0, g1, g2, g3]
  ssems = [s0, s1, s2, s3]
  c = lax.axis_index("c")
  s = lax.axis_index("s")
  ebase = s * EPT
  pltpu.sync_copy(src_w.at[pl.ds(ebase, EPT)], rsrc.at[pl.ds(0, EPT)])
  pltpu.sync_copy(dst_w.at[pl.ds(ebase, EPT)], rdst.at[pl.ds(0, EPT)])
  pltpu.sync_copy(ew_w.at[pl.ds(ebase, EPT)], rew.at[pl.ds(0, EPT)])

  zeros16 = jnp.zeros((16,), jnp.float32)
  iota16 = lax.iota(jnp.int32, 16)
  rsrc[pl.ds(EPT, 16)] = jnp.zeros((16,), jnp.int32)
  rdst[pl.ds(EPT, 16)] = jnp.full((16,), -1, jnp.int32)
  rew[pl.ds(EPT, 16)] = zeros16
  pad16 = jnp.full((16,), EPT, jnp.int32)

  cbase = c * (xs_flat.shape[0] // 2)

  def _prep(b, k, lo):
    for m in range(CH // 16):
      sl = pl.ds(m * 16, 16)
      eid16 = ceid[pl.ds(k * CH + m * 16, 16)]
      d16 = plsc.load_gather(rdst, [eid16]) - lo
      dok = (d16 >= 0) & (d16 < NH)
      dst2[b, sl] = jnp.where(dok, d16, TRASH)
      gidx[b, sl] = plsc.load_gather(rsrc, [eid16]) + cbase
      wbuf[b, sl] = plsc.load_gather(rew, [eid16])

  def _rows(b):
    return rows.at[pl.ds(b * CH, CH)]

  def _issue_g(b):
    pltpu.async_copy(xs_flat.at[gidx.at[b]], _rows(b), gsems[b])

  def _wait_g(b):
    pltpu.make_async_copy(xs_flat.at[gidx.at[b]], _rows(b), gsems[b]).wait()

  def _issue_s(b):
    pltpu.async_copy(_rows(b), acc.at[dst2.at[b]], ssems[b], add=True)

  def _wait_s(b):
    pltpu.make_async_copy(_rows(b), acc.at[dst2.at[b]], ssems[b]).wait()

  def _compute(b):
    def edge_body(m, _):
      w16 = wbuf[b, pl.ds(m * 16, 16)]
      for j in range(16):
        e = b * CH + m * 16 + j
        w = w16[j]
        for f in range(HH // 16):
          sl = pl.ds(f * 16, 16)
          rows[e, sl] = rows[e, sl] * w
      return 0

    lax.fori_loop(0, CH // 16, edge_body, 0)

  def pass_body(p, _):
    lo = p * NH

    def zb_body(i, _):
      for f in range(HH // 16):
        rows[i, pl.ds(f * 16, 16)] = zeros16
      return 0

    lax.fori_loop(0, AZR, zb_body, 0, unroll=2)
    pltpu.sync_copy(rows.at[pl.ds(0, AZR)], acc.at[pl.ds(s * AZR, AZR)])
    plsc.subcore_barrier()

    def comp_body(g, pos):
      sl = pl.ds(g * 16, 16)
      d16 = rdst[sl] - lo
      ok = (d16 >= 0) & (d16 < NH)
      key = jnp.where(ok, iota16, iota16 + 16)
      _, eid = plsc.sort_key_val(key, g * 16 + iota16)
      ceid[pl.ds(pos, 16)] = eid
      cnt = plsc.all_reduce_population_count(ok)
      return pos + cnt[0]

    pos = lax.fori_loop(0, EPT // 16, comp_body, 0, unroll=2)

    for g in range(CH // 16 + 1):
      ceid[pl.ds(pos + g * 16, 16)] = pad16
    nchunks = (pos + CH - 1) // CH

    for b in range(NBUF):
      @pl.when(b < nchunks)
      def _(b=b):
        _prep(b, b, lo)
        _issue_g(b)

    def slot_body(kk, _):
      for b in range(NBUF):
        k = kk * NBUF + b

        @pl.when(k < nchunks)
        def _(b=b, k=k):
          _wait_g(b)
          _compute(b)
          _issue_s(b)

        bn = (b + 2) % NBUF
        kn = k + 2

        @pl.when((kn >= NBUF) & (kn < nchunks))
        def _(b=bn, k=kn):
          _wait_s(b)
          _prep(b, k, lo)
          _issue_g(b)

      return 0

    lax.fori_loop(0, (nchunks + NBUF - 1) // NBUF, slot_body, 0)

    for b in range(NBUF):
      @pl.when(b < nchunks)
      def _(b=b):
        _wait_s(b)

    plsc.subcore_barrier()

    @pl.when(s == 0)
    def _():
      @pl.when(p < NPASS - 1)
      def _():
        pltpu.sync_copy(acc.at[pl.ds(0, NH)],
                        out.at[pl.ds(c * N + lo, NH)])

      @pl.when(p == NPASS - 1)
      def _():
        pltpu.sync_copy(acc.at[pl.ds(0, N - (NPASS - 1) * NH)],
                        out.at[pl.ds(c * N + lo, N - (NPASS - 1) * NH)])

    plsc.subcore_barrier()
    return 0

  lax.fori_loop(0, NPASS, pass_body, 0)


R = 1000
GRID = N // R


def _tc_pre_body(x_ref, w1_ref, degp_ref, xs_ref, dinvr_ref):
  deg = 1.0 + degp_ref[:, 0:1]
  dinv = lax.rsqrt(deg)
  xw = jnp.dot(x_ref[...], w1_ref[...],
               preferred_element_type=jnp.float32)
  xs = xw * dinv
  xs_ref[0] = xs[:, :HH]
  xs_ref[1] = xs[:, HH:]
  dinvr_ref[...] = jnp.broadcast_to(dinv, (R, HH))


def _tc_pre(x, w1, degp):
  return pl.pallas_call(
      _tc_pre_body,
      grid=(GRID,),
      in_specs=[
          pl.BlockSpec((R, F_IN), lambda i: (i, 0)),
          pl.BlockSpec((F_IN, H), lambda i: (0, 0)),
          pl.BlockSpec((R, HH), lambda i: (i, 0)),
      ],
      out_specs=[
          pl.BlockSpec((2, R, HH), lambda i: (0, i, 0)),
          pl.BlockSpec((R, HH), lambda i: (i, 0)),
      ],
      out_shape=[
          jax.ShapeDtypeStruct((2, N, HH), jnp.float32),
          jax.ShapeDtypeStruct((N, HH), jnp.float32),
      ],
  )(x, w1, degp)


def _tc_post_a_body(agg_ref, xs_ref, dinvr_ref, b_ref, a_ref, st_ref):
  i = pl.program_id(0)
  dv = dinvr_ref[:, 0:1]
  left = (agg_ref[0] + xs_ref[0]) * dv
  right = (agg_ref[1] + xs_ref[1]) * dv
  a = jnp.concatenate([left, right], axis=1) + b_ref[...]
  a = jnp.maximum(a, 0.0)
  a_ref[...] = a
  s1 = jnp.sum(a, axis=0, keepdims=True)
  s2 = jnp.sum(a * a, axis=0, keepdims=True)

  @pl.when(i == 0)
  def _():
    st_ref[...] = jnp.zeros((8, H), jnp.float32)

  st_ref[...] += jnp.concatenate(
      [s1, s2, jnp.zeros((6, H), jnp.float32)], axis=0)


def _tc_post_a(agg, xs, dinvr, b):
  return pl.pallas_call(
      _tc_post_a_body,
      grid=(GRID,),
      in_specs=[
          pl.BlockSpec((2, R, HH), lambda i: (0, i, 0)),
          pl.BlockSpec((2, R, HH), lambda i: (0, i, 0)),
          pl.BlockSpec((R, HH), lambda i: (i, 0)),
          pl.BlockSpec((1, H), lambda i: (0, 0)),
      ],
      out_specs=[
          pl.BlockSpec((R, H), lambda i: (i, 0)),
          pl.BlockSpec((8, H), lambda i: (0, 0)),
      ],
      out_shape=[
          jax.ShapeDtypeStruct((N, H), jnp.float32),
          jax.ShapeDtypeStruct((8, H), jnp.float32),
      ],
  )(agg, xs, dinvr, b)


def _bn_from_stats(a, st_ref, g_ref, be_ref):
  m = st_ref[0:1, :] * (1.0 / N)
  ex2 = st_ref[1:2, :] * (1.0 / N)
  var = ex2 - m * m
  return (a - m) * lax.rsqrt(var + 1e-5) * g_ref[...] + be_ref[...]


def _tc_post_b_body(a_ref, st_ref, g_ref, be_ref, w2_ref, dinvr_ref,
                    h1_ref, xs2_ref):
  h1 = _bn_from_stats(a_ref[...], st_ref, g_ref, be_ref)
  h1_ref[...] = h1
  h1w = jnp.dot(h1, w2_ref[...], preferred_element_type=jnp.float32)
  xs2 = h1w * dinvr_ref[:, 0:1]
  xs2_ref[0] = xs2[:, :HH]
  xs2_ref[1] = xs2[:, HH:]


def _tc_post_b(a, st, g, be, w2, dinvr):
  return pl.pallas_call(
      _tc_post_b_body,
      grid=(GRID,),
      in_specs=[
          pl.BlockSpec((R, H), lambda i: (i, 0)),
          pl.BlockSpec((8, H), lambda i: (0, 0)),
          pl.BlockSpec((1, H), lambda i: (0, 0)),
          pl.BlockSpec((1, H), lambda i: (0, 0)),
          pl.BlockSpec((H, H), lambda i: (0, 0)),
          pl.BlockSpec((R, HH), lambda i: (i, 0)),
      ],
      out_specs=[
          pl.BlockSpec((R, H), lambda i: (i, 0)),
          pl.BlockSpec((2, R, HH), lambda i: (0, i, 0)),
      ],
      out_shape=[
          jax.ShapeDtypeStruct((N, H), jnp.float32),
          jax.ShapeDtypeStruct((2, N, HH), jnp.float32),
      ],
  )(a, st, g, be, w2, dinvr)


def _tc_final_body(a2_ref, st_ref, g_ref, be_ref, h1_ref, x_ref,
                   wih1t_ref, bs1_ref, wih2t_ref, bs2_ref, wlt_ref, bl_ref,
                   out_ref):
  h2 = _bn_from_stats(a2_ref[...], st_ref, g_ref, be_ref)
  xc = jnp.concatenate([h1_ref[...], h2], axis=1)
  gates = jnp.dot(xc, wih1t_ref[...],
                  preferred_element_type=jnp.float32) + bs1_ref[...]
  ig = jax.nn.sigmoid(gates[:, :H])
  gg = jnp.tanh(gates[:, 2 * H:3 * H])
  og = jax.nn.sigmoid(gates[:, 3 * H:])
  hn1 = og * jnp.tanh(ig * gg)
  gates2 = jnp.dot(hn1, wih2t_ref[...],
                   preferred_element_type=jnp.float32) + bs2_ref[...]
  ig2 = jax.nn.sigmoid(gates2[:, :H])
  gg2 = jnp.tanh(gates2[:, 2 * H:3 * H])
  og2 = jax.nn.sigmoid(gates2[:, 3 * H:])
  hn2 = og2 * jnp.tanh(ig2 * gg2)
  hcat = jnp.concatenate([hn1, hn2, x_ref[...]], axis=1)
  hcat = jnp.maximum(hcat, 0.0)
  out_ref[...] = jnp.dot(hcat, wlt_ref[...],
                         preferred_element_type=jnp.float32) + bl_ref[...]


def _tc_final(a2, st, g, be, h1, x, wih1t, bs1, wih2t, bs2, wlt, bl):
  din = 2 * H + F_IN
  return pl.pallas_call(
      _tc_final_body,
      grid=(GRID,),
      in_specs=[
          pl.BlockSpec((R, H), lambda i: (i, 0)),
          pl.BlockSpec((8, H), lambda i: (0, 0)),
          pl.BlockSpec((1, H), lambda i: (0, 0)),
          pl.BlockSpec((1, H), lambda i: (0, 0)),
          pl.BlockSpec((R, H), lambda i: (i, 0)),
          pl.BlockSpec((R, F_IN), lambda i: (i, 0)),
          pl.BlockSpec((2 * H, 4 * H), lambda i: (0, 0)),
          pl.BlockSpec((1, 4 * H), lambda i: (0, 0)),
          pl.BlockSpec((H, 4 * H), lambda i: (0, 0)),
          pl.BlockSpec((1, 4 * H), lambda i: (0, 0)),
          pl.BlockSpec((din, T_OUT), lambda i: (0, 0)),
          pl.BlockSpec((1, T_OUT), lambda i: (0, 0)),
      ],
      out_specs=pl.BlockSpec((R, T_OUT), lambda i: (i, 0)),
      out_shape=jax.ShapeDtypeStruct((N, T_OUT), jnp.float32),
  )(a2, st, g, be, h1, x, wih1t, bs1, wih2t, bs2, wlt, bl)


def kernel(x, edge_index, edge_weight, W1, b1, g1, be1, W2, b2, g2, be2,
           Wih1, Whh1, bih1, bhh1, Wih2, Whh2, bih2, bhh2, Wl, bl):
  src = edge_index[0]
  dst = edge_index[1]
  ew = edge_weight

  ones_tab = jnp.ones((2 * N, HH), jnp.float32)
  degp = _sc_agg(ones_tab, src, dst, ew)

  xs1, dinvr = _tc_pre(x, W1, degp)
  agg1 = _sc_agg(xs1.reshape(2 * N, HH), src, dst, ew)
  a1, st1 = _tc_post_a(agg1.reshape(2, N, HH), xs1, dinvr,
                       b1.reshape(1, H))
  h1, xs2 = _tc_post_b(a1, st1, g1.reshape(1, H), be1.reshape(1, H), W2,
                       dinvr)
  agg2 = _sc_agg(xs2.reshape(2 * N, HH), src, dst, ew)
  a2, st2 = _tc_post_a(agg2.reshape(2, N, HH), xs2, dinvr,
                       b2.reshape(1, H))
  out = _tc_final(a2, st2, g2.reshape(1, H), be2.reshape(1, H), h1, x,
                  Wih1.T, (bih1 + bhh1).reshape(1, 4 * H),
                  Wih2.T, (bih2 + bhh2).reshape(1, 4 * H),
                  Wl.T, bl.reshape(1, T_OUT))
  return out
